# baseline (speedup 1.0000x reference)
import functools

import jax
import jax.numpy as jnp
from jax import lax
from jax.experimental import pallas as pl
from jax.experimental.pallas import tpu as pltpu
from jax.experimental.pallas import tpu_sc as plsc

N = 50000
NP = 53248
G = 64
GC = 32
G3 = G * G * G
GC3 = GC * GC * GC
G3P = G3 + 4096
GC3P = GC3 + 4096
SENT = N
DUMF = G3
DUMC = GC3
CHUNK = 128

_SC_PARAMS = pltpu.CompilerParams(use_tc_tiling_on_sc=False)


@functools.lru_cache(maxsize=1)
def _sc_mesh():
    return plsc.VectorSubcoreMesh(
        core_axis_name="c", subcore_axis_name="s",
        num_cores=2, num_subcores=16)

_i32 = jnp.int32
_f32 = jnp.float32


def _iota16():
    return lax.iota(_i32, 16)


def _build_geo_sums(cT, P, sentf, sentc):
    rows_t = NP // 16
    nchunks = rows_t // CHUNK
    gf_t = G3P // 16
    gc_t = GC3P // 16

    @functools.partial(
        pl.kernel,
        out_type=(
            jax.ShapeDtypeStruct((G3P,), _i32),
            jax.ShapeDtypeStruct((GC3P,), _i32),
            jax.ShapeDtypeStruct((NP, 16), _f32),
        ),
        mesh=_sc_mesh(),
        compiler_params=_SC_PARAMS,
        scratch_types=[
            pltpu.VMEM_SHARED((NP, 16), _f32),
            pltpu.VMEM((CHUNK,), _i32),
            pltpu.VMEM((CHUNK,), _i32),
            pltpu.VMEM((CHUNK,), _i32),
            pltpu.VMEM((CHUNK,), _i32),
            pltpu.VMEM((CHUNK,), _i32),
            pltpu.VMEM((CHUNK,), _i32),
            pltpu.VMEM((CHUNK, 16), _f32),
            pltpu.SemaphoreType.DMA,
        ],
    )
    def k(cx_h, cy_h, cz_h, P_h, sentf_h, sentc_h, zacc_h,
          gridf_o, gridc_o, sums_o,
          acc_s, xb, yb, zb, ffb, fcb, valsb, prow, sem):
        cid = lax.axis_index("c")
        sid = lax.axis_index("s")

        @pl.when(cid == 0)
        def _():
            t = sid
            pltpu.sync_copy(sentf_h.at[pl.ds(t * gf_t, gf_t)],
                            gridf_o.at[pl.ds(t * gf_t, gf_t)])
            pltpu.sync_copy(sentc_h.at[pl.ds(t * gc_t, gc_t)],
                            gridc_o.at[pl.ds(t * gc_t, gc_t)])
            pltpu.sync_copy(zacc_h.at[pl.ds(t * rows_t, rows_t)],
                            acc_s.at[pl.ds(t * rows_t, rows_t)])
            plsc.subcore_barrier()

            def load_flat(base):
                pltpu.sync_copy(cx_h.at[pl.ds(base, CHUNK)], xb)
                pltpu.sync_copy(cy_h.at[pl.ds(base, CHUNK)], yb)
                pltpu.sync_copy(cz_h.at[pl.ds(base, CHUNK)], zb)
                for j in range(CHUNK // 16):
                    s = pl.ds(j * 16, 16)
                    xv, yv, zv = xb[s], yb[s], zb[s]
                    ff = (xv * G + yv) * G + zv
                    okf = (ff >= 0) & (ff < G3)
                    ffb[s] = jnp.where(okf, ff, DUMF)
                    xq, yq, zq = xv >> 1, yv >> 1, zv >> 1
                    fc = (xq * GC + yq) * GC + zq
                    okc = (fc >= 0) & (fc < GC3)
                    fcb[s] = jnp.where(okc, fc, DUMC)

            def chunk_b(ci, _):
                base = t * rows_t + ci * CHUNK
                load_flat(base)
                for j in range(CHUNK // 16):
                    s = pl.ds(j * 16, 16)
                    valsb[s] = base + j * 16 + _iota16()
                pltpu.sync_copy(valsb, gridf_o.at[ffb])
                pltpu.sync_copy(valsb, gridc_o.at[fcb])
                return _

            lax.fori_loop(0, nchunks, chunk_b, None)
            plsc.subcore_barrier()

            def chunk_c(ci, _):
                base = t * rows_t + ci * CHUNK
                load_flat(base)
                pltpu.async_copy(gridf_o.at[ffb], valsb, sem).wait()
                pltpu.sync_copy(P_h.at[pl.ds(base, CHUNK)], prow)
                pltpu.sync_copy(prow, acc_s.at[valsb], add=True)
                return _

            lax.fori_loop(0, nchunks, chunk_c, None)
            plsc.subcore_barrier()

            pltpu.sync_copy(acc_s.at[pl.ds(t * rows_t, rows_t)],
                            sums_o.at[pl.ds(t * rows_t, rows_t)])

    zacc = jnp.zeros((NP, 16), _f32)
    return k(cT[0], cT[1], cT[2], P, sentf, sentc, zacc)


_SC_TILED = pltpu.CompilerParams(use_tc_tiling_on_sc=True,
                                 needs_layout_passes=False)

_NIDF_K = 27
_NIDC_K = 27
_NIDK_K = 8


def _make_nid():
    rows_t = NP // 32
    nchunks = rows_t // CHUNK
    NOFF = _NIDF_K + _NIDC_K + _NIDK_K

    @functools.partial(
        pl.kernel,
        out_type=(
            jax.ShapeDtypeStruct((_NIDF_K, NP), _i32),
            jax.ShapeDtypeStruct((_NIDC_K, NP), _i32),
            jax.ShapeDtypeStruct((_NIDK_K, NP), _i32),
        ),
        mesh=_sc_mesh(),
        compiler_params=_SC_TILED,
        scratch_types=(
            [pltpu.VMEM((CHUNK,), _i32)] * 3
            + [pltpu.VMEM((NOFF, CHUNK), _i32)] * 2
            + [pltpu.VMEM((CHUNK, 128), _i32)] * 2
            + [pltpu.VMEM((NOFF, CHUNK), _i32)]
            + [pltpu.SemaphoreType.DMA]
        ),
    )
    def k(cx_h, cy_h, cz_h, gf_h, gc_h, nidf_o, nidc_o, nidk_o,
          xb, yb, zb, fl2d, rw2d, rb0, rb1, nsl, semg):
        rowbufs = [rb0, rb1]
        cid = lax.axis_index("c")
        sid = lax.axis_index("s")
        wid = sid * 2 + cid
        row0 = wid * rows_t

        def chunk(ci, _):
            base = row0 + ci * CHUNK
            pltpu.sync_copy(cx_h.at[pl.ds(base, CHUNK)], xb)
            pltpu.sync_copy(cy_h.at[pl.ds(base, CHUNK)], yb)
            pltpu.sync_copy(cz_h.at[pl.ds(base, CHUNK)], zb)
            for j in range(CHUNK // 16):
                s = pl.ds(j * 16, 16)
                xv, yv, zv = xb[s], yb[s], zb[s]
                f0 = (xv * G + yv) * G + zv
                mx = {d: (xv + d >= 0) & (xv + d < G) for d in (-1, 0, 1)}
                my = {d: (yv + d >= 0) & (yv + d < G) for d in (-1, 0, 1)}
                mz = {d: (zv + d >= 0) & (zv + d < G) for d in (-1, 0, 1)}
                for kk in range(27):
                    dx, dy, dz = kk // 9 - 1, (kk // 3) % 3 - 1, kk % 3 - 1
                    dk = (dx * G + dy) * G + dz
                    inb = mx[dx] & my[dy] & mz[dz]
                    fl = jnp.where(inb, f0 + dk, DUMF)
                    fl2d[kk, s] = fl
                    rw2d[kk, s] = fl >> 7
                xq, yq, zq = xv >> 1, yv >> 1, zv >> 1
                f0c = (xq * GC + yq) * GC + zq
                mxc = {d: (xq + d >= 0) & (xq + d < GC) for d in (-1, 0, 1)}
                myc = {d: (yq + d >= 0) & (yq + d < GC) for d in (-1, 0, 1)}
                mzc = {d: (zq + d >= 0) & (zq + d < GC) for d in (-1, 0, 1)}
                for kk in range(27):
                    dx, dy, dz = kk // 9 - 1, (kk // 3) % 3 - 1, kk % 3 - 1
                    dk = (dx * GC + dy) * GC + dz
                    inb = mxc[dx] & myc[dy] & mzc[dz]
                    fl = jnp.where(inb, f0c + dk, DUMC)
                    fl2d[27 + kk, s] = fl
                    rw2d[27 + kk, s] = fl >> 7
                bx2, by2, bz2 = xq << 1, yq << 1, zq << 1
                f0k = (bx2 * G + by2) * G + bz2
                mok = ((bx2 >= 0) & (bx2 < G) & (by2 >= 0) & (by2 < G)
                       & (bz2 >= 0) & (bz2 < G))
                for kk in range(8):
                    dx, dy, dz = (kk >> 2) & 1, (kk >> 1) & 1, kk & 1
                    dk = (dx * G + dy) * G + dz
                    fl = jnp.where(mok, f0k + dk, DUMF)
                    fl2d[54 + kk, s] = fl
                    rw2d[54 + kk, s] = fl >> 7

            def grid_of(o):
                return gf_h if (o < 27 or o >= 54) else gc_h

            def process(o, rb):
                for j in range(CHUNK // 16):
                    s = pl.ds(j * 16, 16)
                    rowv = j * 16 + _iota16()
                    colv = fl2d[o, s] & 127
                    nsl[o, s] = plsc.load_gather(rb, [rowv, colv])

            descs = {}
            for o in range(NOFF):
                if o >= 2:
                    descs[o - 2].wait()
                    process(o - 2, rowbufs[o % 2])
                descs[o] = pltpu.async_copy(
                    grid_of(o).at[rw2d.at[o]], rowbufs[o % 2], semg)
            for o in (NOFF - 2, NOFF - 1):
                descs[o].wait()
                process(o, rowbufs[o % 2])
            pltpu.sync_copy(nsl.at[pl.ds(0, 27)],
                            nidf_o.at[:, pl.ds(base, CHUNK)])
            pltpu.sync_copy(nsl.at[pl.ds(27, 27)],
                            nidc_o.at[:, pl.ds(base, CHUNK)])
            pltpu.sync_copy(nsl.at[pl.ds(54, 8)],
                            nidk_o.at[:, pl.ds(base, CHUNK)])
            return _

        lax.fori_loop(0, nchunks, chunk, None)

    return k


def _make_rowgather(C, K):
    CH = CHUNK
    rows_t = NP // 32
    nchunks = rows_t // CH
    W = 1 if C > 128 else 3
    W = min(W, K)
    NB = 2 * W

    @functools.partial(
        pl.kernel,
        out_type=jax.ShapeDtypeStruct((K, NP, C), _f32),
        mesh=_sc_mesh(),
        compiler_params=_SC_TILED,
        scratch_types=(
            [pltpu.VMEM((K, CH), _i32)]
            + [pltpu.VMEM((CH, C), _f32)] * NB
            + [pltpu.SemaphoreType.DMA] * 3
        ),
    )
    def k(nid_h, tbl_h, out_h, nid2d, *rest):
        bufs = list(rest[:NB])
        semn, semr, semw = rest[NB:]
        cid = lax.axis_index("c")
        sid = lax.axis_index("s")
        wid = sid * 2 + cid
        row0 = wid * rows_t

        def chunk(ci, _):
            base = row0 + ci * CH
            pltpu.sync_copy(nid_h.at[:, pl.ds(base, CH)], nid2d)
            rds, wds = {}, {}
            for kk in range(K):
                if kk >= NB:
                    wds[kk - NB].wait()
                rds[kk] = pltpu.async_copy(tbl_h.at[nid2d.at[kk]],
                                           bufs[kk % NB], semr)
                if kk >= W:
                    kw = kk - W
                    rds[kw].wait()
                    wds[kw] = pltpu.async_copy(
                        bufs[kw % NB], out_h.at[kw, pl.ds(base, CH)], semw)
            for kw in range(max(0, K - W), K):
                rds[kw].wait()
                wds[kw] = pltpu.async_copy(
                    bufs[kw % NB], out_h.at[kw, pl.ds(base, CH)], semw)
            for kw in range(max(0, K - NB), K):
                wds[kw].wait()
            return _

        lax.fori_loop(0, nchunks, chunk, None)

    return k


def _make_up_gather(C):
    rows_t = NP // 32
    nchunks = rows_t // CHUNK

    @functools.partial(
        pl.kernel,
        out_type=jax.ShapeDtypeStruct((NP, C), _f32),
        mesh=_sc_mesh(),
        compiler_params=_SC_TILED,
        scratch_types=[
            pltpu.VMEM((CHUNK,), _i32),
            pltpu.VMEM((CHUNK,), _i32),
            pltpu.VMEM((CHUNK,), _i32),
            pltpu.VMEM((CHUNK,), _i32),
            pltpu.VMEM((CHUNK, C), _f32),
            pltpu.VMEM((CHUNK, C), _f32),
            pltpu.SemaphoreType.DMA,
            pltpu.SemaphoreType.DMA,
        ],
    )
    def k(cx_h, cy_h, cz_h, tbl_h, out_h, xb, yb, zb, idxb, r0, r1, semr,
          semw):
        rows = [r0, r1]
        cid = lax.axis_index("c")
        sid = lax.axis_index("s")
        wid = sid * 2 + cid
        row0 = wid * rows_t

        def chunk(ci, _):
            base = row0 + ci * CHUNK
            pltpu.sync_copy(cx_h.at[pl.ds(base, CHUNK)], xb)
            pltpu.sync_copy(cy_h.at[pl.ds(base, CHUNK)], yb)
            pltpu.sync_copy(cz_h.at[pl.ds(base, CHUNK)], zb)
            for j in range(CHUNK // 16):
                s = pl.ds(j * 16, 16)
                oidx = ((xb[s] & 1) * 2 + (yb[s] & 1)) * 2 + (zb[s] & 1)
                idxb[s] = (base + j * 16 + _iota16()) * 8 + oidx
            pltpu.async_copy(tbl_h.at[idxb], rows[0], semr).wait()
            pltpu.async_copy(rows[0], out_h.at[pl.ds(base, CHUNK)],
                             semw).wait()
            return _

        lax.fori_loop(0, nchunks, chunk, None)

    return k


BN = 512
NBLK = NP // BN

_TC_PARAMS = pltpu.CompilerParams(
    dimension_semantics=("parallel", "arbitrary"))
_TC_PARAMS1 = pltpu.CompilerParams(dimension_semantics=("parallel",))


def _bspec_g(BNr, C):
    return pl.BlockSpec((1, BNr, C), lambda i, k: (k, i, 0))


def _bspec_w(C, D):
    return pl.BlockSpec((1, C, D), lambda i, k: (k, 0, 0))


def _bspec_row(BNr, C):
    return pl.BlockSpec((BNr, C), lambda i, k: (i, 0))


def _bspec_vec(D):
    return pl.BlockSpec((1, D), lambda i, k: (0, 0))


def _tc_conv_plain(Gt, W, g, b, relu):
    K, _, C = Gt.shape
    D = W.shape[2]

    def body(G_ref, W_ref, g_ref, b_ref, out_ref):
        k = pl.program_id(1)
        acc = jnp.dot(G_ref[0], W_ref[0], preferred_element_type=_f32)

        @pl.when(k == 0)
        def _():
            out_ref[...] = acc

        @pl.when(k > 0)
        def _():
            out_ref[...] += acc

        @pl.when(k == K - 1)
        def _():
            y = out_ref[...] * g_ref[...] + b_ref[...]
            out_ref[...] = jnp.maximum(y, 0.0) if relu else y

    return pl.pallas_call(
        body,
        grid=(NBLK, K),
        in_specs=[_bspec_g(BN, C), _bspec_w(C, D), _bspec_vec(D), _bspec_vec(D)],
        out_specs=_bspec_row(BN, D),
        out_shape=jax.ShapeDtypeStruct((NP, D), _f32),
        compiler_params=_TC_PARAMS,
    )(Gt, W, g, b)


def _tc_conv_res(Gt, W, g, b, Xd, Wd, gd, bd):
    K, _, C = Gt.shape
    D = W.shape[2]
    Cd = Xd.shape[1]

    def body(G_ref, W_ref, g_ref, b_ref, X_ref, Wd_ref, gd_ref, bd_ref,
             out_ref):
        k = pl.program_id(1)
        acc = jnp.dot(G_ref[0], W_ref[0], preferred_element_type=_f32)

        @pl.when(k == 0)
        def _():
            out_ref[...] = acc

        @pl.when(k > 0)
        def _():
            out_ref[...] += acc

        @pl.when(k == K - 1)
        def _():
            d = jnp.dot(X_ref[...], Wd_ref[...], preferred_element_type=_f32)
            y = (out_ref[...] * g_ref[...] + b_ref[...]
                 + d * gd_ref[...] + bd_ref[...])
            out_ref[...] = jnp.maximum(y, 0.0)

    return pl.pallas_call(
        body,
        grid=(NBLK, K),
        in_specs=[_bspec_g(BN, C), _bspec_w(C, D), _bspec_vec(D),
                  _bspec_vec(D), _bspec_row(BN, Cd),
                  pl.BlockSpec((Cd, D), lambda i, k: (0, 0)),
                  _bspec_vec(D), _bspec_vec(D)],
        out_specs=_bspec_row(BN, D),
        out_shape=jax.ShapeDtypeStruct((NP, D), _f32),
        compiler_params=_TC_PARAMS,
    )(Gt, W, g, b, Xd, Wd, gd, bd)


def _tc_conv_dual(Gt, W1, g1, b1, W2, g2, b2):
    K, _, C = Gt.shape
    D = W1.shape[2]

    def body(G_ref, W1_ref, g1_ref, b1_ref, W2_ref, g2_ref, b2_ref,
             out_ref, acc2):
        k = pl.program_id(1)
        S = G_ref[0]
        V = S / jnp.maximum(S[:, 0:1], 1.0)
        a1 = jnp.dot(V, W1_ref[0], preferred_element_type=_f32)
        a2 = jnp.dot(V, W2_ref[0], preferred_element_type=_f32)

        @pl.when(k == 0)
        def _():
            out_ref[...] = a1
            acc2[...] = a2

        @pl.when(k > 0)
        def _():
            out_ref[...] += a1
            acc2[...] += a2

        @pl.when(k == K - 1)
        def _():
            y1 = jnp.maximum(out_ref[...] * g1_ref[...] + b1_ref[...], 0.0)
            y2 = jnp.maximum(acc2[...] * g2_ref[...] + b2_ref[...], 0.0)
            out_ref[...] = y1 + y2

    return pl.pallas_call(
        body,
        grid=(NBLK, K),
        in_specs=[_bspec_g(BN, C), _bspec_w(C, D), _bspec_vec(D),
                  _bspec_vec(D), _bspec_w(C, D), _bspec_vec(D), _bspec_vec(D)],
        out_specs=_bspec_row(BN, D),
        out_shape=jax.ShapeDtypeStruct((NP, D), _f32),
        scratch_shapes=[pltpu.VMEM((BN, D), _f32)],
        compiler_params=_TC_PARAMS,
    )(Gt, W1, g1, b1, W2, g2, b2)


def _tc_dense_pack(x1, x0, sums, W, bias, g, b):
    C = x0.shape[1]
    D = x1.shape[1]

    def body(x1_ref, x0_ref, s_ref, W_ref, bias_ref, g_ref, b_ref,
             z1_ref, tz_ref):
        f0 = jnp.dot(x0_ref[...], W_ref[...], preferred_element_type=_f32)
        f0 = jnp.maximum((f0 + bias_ref[...]) * g_ref[...] + b_ref[...], 0.0)
        z1 = x1_ref[...] + f0
        z1_ref[...] = z1
        cnt = s_ref[:, 0:1]
        tz_ref[...] = jnp.concatenate(
            [cnt, jnp.zeros((z1.shape[0], 7), _f32), cnt * z1,
             jnp.zeros((z1.shape[0], 120), _f32)], axis=1)

    return pl.pallas_call(
        body,
        grid=(NBLK,),
        in_specs=[pl.BlockSpec((BN, D), lambda i: (i, 0)),
                  pl.BlockSpec((BN, C), lambda i: (i, 0)),
                  pl.BlockSpec((BN, 16), lambda i: (i, 0)),
                  pl.BlockSpec((C, D), lambda i: (0, 0)),
                  pl.BlockSpec((1, D), lambda i: (0, 0)),
                  pl.BlockSpec((1, D), lambda i: (0, 0)),
                  pl.BlockSpec((1, D), lambda i: (0, 0))],
        out_specs=[pl.BlockSpec((BN, D), lambda i: (i, 0)),
                   pl.BlockSpec((BN, D + 128), lambda i: (i, 0))],
        out_shape=[jax.ShapeDtypeStruct((NP, D), _f32),
                   jax.ShapeDtypeStruct((NP, D + 128), _f32)],
        compiler_params=_TC_PARAMS1,
    )(x1, x0, sums, W, bias, g, b)


def _tc_y2c_up(Gz, upW):
    K = Gz.shape[0]
    Ct = Gz.shape[2]
    D = upW.shape[2]

    def body(G_ref, W_ref, out_ref):
        acc = G_ref[0]
        for k in range(1, K):
            acc = acc + G_ref[k]
        y2c = acc[:, 8:136] / jnp.maximum(acc[:, 0:1], 1.0)
        outs = [jnp.dot(y2c, W_ref[k], preferred_element_type=_f32)
                for k in range(8)]
        out_ref[...] = jnp.concatenate(outs, axis=1)

    return pl.pallas_call(
        body,
        grid=(NBLK,),
        in_specs=[pl.BlockSpec((K, BN, Ct), lambda i: (0, i, 0)),
                  pl.BlockSpec((8, 128, D), lambda i: (0, 0, 0))],
        out_specs=pl.BlockSpec((BN, 8 * D), lambda i: (i, 0)),
        out_shape=jax.ShapeDtypeStruct((NP, 8 * D), _f32),
        compiler_params=_TC_PARAMS1,
    )(Gz, upW)


def _tc_upcat(Gu, g, b, x0):
    C = Gu.shape[1]
    C2 = x0.shape[1]

    def body(Gu_ref, g_ref, b_ref, x0_ref, out_ref):
        y = jnp.maximum(Gu_ref[...] * g_ref[...] + b_ref[...], 0.0)
        out_ref[...] = jnp.concatenate([y, x0_ref[...]], axis=1)

    return pl.pallas_call(
        body,
        grid=(NBLK,),
        in_specs=[pl.BlockSpec((BN, C), lambda i: (i, 0)),
                  pl.BlockSpec((1, C), lambda i: (0, 0)),
                  pl.BlockSpec((1, C), lambda i: (0, 0)),
                  pl.BlockSpec((BN, C2), lambda i: (i, 0))],
        out_specs=pl.BlockSpec((BN, C + C2), lambda i: (i, 0)),
        out_shape=jax.ShapeDtypeStruct((NP, C + C2), _f32),
        compiler_params=_TC_PARAMS1,
    )(Gu, g, b, x0)


def _tc_final(G4, W, g, b, y2cat, Wd, gd, bd, z1, Wp, biasp, gp, bp,
              occW, occb):
    K, _, C = G4.shape
    D = W.shape[2]
    Cd = y2cat.shape[1]
    Cp = z1.shape[1]
    Do = occW.shape[1]

    def body(G_ref, W_ref, g_ref, b_ref, yc_ref, Wd_ref, gd_ref, bd_ref,
             z1_ref, Wp_ref, biasp_ref, gp_ref, bp_ref, occW_ref, occb_ref,
             out_ref, acc):
        k = pl.program_id(1)
        a = jnp.dot(G_ref[0], W_ref[0], preferred_element_type=_f32)

        @pl.when(k == 0)
        def _():
            acc[...] = a

        @pl.when(k > 0)
        def _():
            acc[...] += a

        @pl.when(k == K - 1)
        def _():
            d = jnp.dot(yc_ref[...], Wd_ref[...], preferred_element_type=_f32)
            y2f = jnp.maximum(acc[...] * g_ref[...] + b_ref[...]
                              + d * gd_ref[...] + bd_ref[...], 0.0)
            f1 = jnp.dot(z1_ref[...], Wp_ref[...], preferred_element_type=_f32)
            f1 = jnp.maximum((f1 + biasp_ref[...]) * gp_ref[...]
                             + bp_ref[...], 0.0)
            z2 = y2f + f1
            out_ref[...] = jnp.dot(z2, occW_ref[...],
                                   preferred_element_type=_f32) + occb_ref[...]

    return pl.pallas_call(
        body,
        grid=(NBLK, K),
        in_specs=[_bspec_g(BN, C), _bspec_w(C, D), _bspec_vec(D),
                  _bspec_vec(D), _bspec_row(BN, Cd),
                  pl.BlockSpec((Cd, D), lambda i, k: (0, 0)),
                  _bspec_vec(D), _bspec_vec(D), _bspec_row(BN, Cp),
                  pl.BlockSpec((Cp, D), lambda i, k: (0, 0)),
                  _bspec_vec(D), _bspec_vec(D), _bspec_vec(D),
                  pl.BlockSpec((D, Do), lambda i, k: (0, 0)),
                  pl.BlockSpec((1, Do), lambda i, k: (0, 0))],
        out_specs=_bspec_row(BN, Do),
        out_shape=jax.ShapeDtypeStruct((NP, Do), _f32),
        scratch_shapes=[pltpu.VMEM((BN, D), _f32)],
        compiler_params=_TC_PARAMS,
    )(G4, W, g, b, y2cat, Wd, gd, bd, z1, Wp, biasp, gp, bp, occW, occb)


def _tc_widen(s):
    def body(s_ref, out_ref):
        out_ref[...] = jnp.concatenate(
            [s_ref[...], jnp.zeros((s_ref.shape[0], 112), _f32)], axis=1)

    return pl.pallas_call(
        body,
        grid=(NBLK,),
        in_specs=[pl.BlockSpec((BN, 16), lambda i: (i, 0))],
        out_specs=pl.BlockSpec((BN, 128), lambda i: (i, 0)),
        out_shape=jax.ShapeDtypeStruct((NP, 128), _f32),
        compiler_params=_TC_PARAMS1,
    )(s)


def kernel(z_depth, x_feat_raw, z_feat_raw, coords, pres, vres, params):
    p = params
    n = coords.shape[0]

    cT = jnp.full((3, NP), -64, _i32).at[:, :n].set(coords.T.astype(_i32))
    cx, cy, cz = cT[0], cT[1], cT[2]
    P = jnp.zeros((NP, 16), _f32)
    P = P.at[:n, 0].set(1.0)
    P = P.at[:n, 1].set(z_depth[:, 0])
    P = P.at[:n, 2:6].set(x_feat_raw)
    sentf = jnp.full((G3P,), SENT, _i32)
    sentc = jnp.full((GC3P,), SENT, _i32)

    def pv(x, fill):
        out = jnp.full((1, 128), fill, _f32)
        return out.at[0, :x.shape[0]].set(x)

    def pw(Wst, R, roff=0):
        K = Wst.shape[0]
        out = jnp.zeros((K, R, 128), _f32)
        return out.at[:, roff:roff + Wst.shape[1], :Wst.shape[2]].set(Wst)

    def pw2(Wm, R, roff=0):
        out = jnp.zeros((R, 128), _f32)
        return out.at[roff:roff + Wm.shape[0], :Wm.shape[1]].set(Wm)

    Wd0 = jnp.zeros((27, 128, 128), _f32).at[:, 1, :96].set(p['sd_W'][:, 0, :])
    Wf0 = jnp.zeros((27, 128, 128), _f32).at[:, 2:6, :96].set(p['sf_W'])

    gridf, gridc, sums16 = _build_geo_sums(cT, P, sentf, sentc)
    gf2 = gridf.reshape(G3P // 128, 128)
    gc2 = gridc.reshape(GC3P // 128, 128)
    nidf, nidc, nidk = _make_nid()(cx, cy, cz, gf2, gc2)
    sums = _tc_widen(sums16)

    G0 = _make_rowgather(128, 27)(nidf, sums)
    x0 = _tc_conv_dual(G0, Wd0, pv(p['sd_g'], 1.0), pv(p['sd_b'], 0.0),
                       Wf0, pv(p['sf_g'], 1.0), pv(p['sf_b'], 0.0))

    Gd = _make_rowgather(128, 8)(nidk, x0)
    x1c = _tc_conv_plain(Gd, pw(p['st_W'], 128),
                         pv(p['st_g'], 1.0), pv(p['st_b'], 0.0), True)

    G1 = _make_rowgather(128, 27)(nidc, x1c)
    h = _tc_conv_plain(G1, pw(p['r1a_W'], 128),
                       pv(p['r1a_g'], 1.0), pv(p['r1a_b'], 0.0), True)
    G2 = _make_rowgather(128, 27)(nidc, h)
    x1 = _tc_conv_res(G2, pw(p['r1b_W'], 128),
                      pv(p['r1b_g'], 1.0), pv(p['r1b_b'], 0.0),
                      x1c, pw2(p['r1d_W'], 128),
                      pv(p['r1d_g'], 1.0), pv(p['r1d_b'], 0.0))

    z1, Tz = _tc_dense_pack(x1, x0, sums16, pw2(p['pt0_W'], 128),
                            pv(p['pt0_bias'], 0.0),
                            pv(p['pt0_g'], 1.0), pv(p['pt0_b'], 0.0))
    Gz = _make_rowgather(256, 8)(nidk, Tz)
    Yup = _tc_y2c_up(Gz, pw(p['up_W'], 128))

    Gu = _make_up_gather(128)(cx, cy, cz, Yup.reshape(8 * NP, 128))
    y2cat = _tc_upcat(Gu, pv(jnp.pad(p['up_g'], (0, 80)), 1.0),
                      pv(p['up_b'], 0.0), x0)

    r2aW = jnp.zeros((27, 256, 128), _f32)
    r2aW = r2aW.at[:, 0:48, :48].set(p['r2a_W'][:, 0:48, :])
    r2aW = r2aW.at[:, 128:224, :48].set(p['r2a_W'][:, 48:144, :])
    G3g = _make_rowgather(256, 27)(nidf, y2cat)
    h2 = _tc_conv_plain(G3g, r2aW, pv(p['r2a_g'], 1.0), pv(p['r2a_b'], 0.0),
                        True)
    G4 = _make_rowgather(128, 27)(nidf, h2)
    r2dW = jnp.zeros((256, 128), _f32)
    r2dW = r2dW.at[0:48, :48].set(p['r2d_W'][0:48, :])
    r2dW = r2dW.at[128:224, :48].set(p['r2d_W'][48:144, :])
    occW = jnp.zeros((128, 8), _f32).at[:48, 0].set(p['occ_W'][:, 0])
    occb = jnp.zeros((1, 8), _f32).at[0, 0].set(p['occ_b'][0])
    out = _tc_final(G4, pw(p['r2b_W'], 128),
                    pv(p['r2b_g'], 1.0), pv(p['r2b_b'], 0.0),
                    y2cat, r2dW, pv(p['r2d_g'], 1.0), pv(p['r2d_b'], 0.0),
                    z1, pw2(p['pt1_W'], 128), pv(p['pt1_bias'], 0.0),
                    pv(p['pt1_g'], 1.0), pv(p['pt1_b'], 0.0), occW, occb)
    return out[:n, :1]


# speedup vs baseline: 1.6298x; 1.6298x over previous
import functools

import jax
import jax.numpy as jnp
from jax import lax
from jax.experimental import pallas as pl
from jax.experimental.pallas import tpu as pltpu
from jax.experimental.pallas import tpu_sc as plsc

N = 50000
NP = 53248
G = 64
GC = 32
G3 = G * G * G
GC3 = GC * GC * GC
G3P = G3 + 4096
GC3P = GC3 + 4096
SENT = N
DUMF = G3
DUMC = GC3
CHUNK = 128

_SC_PARAMS = pltpu.CompilerParams(use_tc_tiling_on_sc=False)


@functools.lru_cache(maxsize=1)
def _sc_mesh():
    return plsc.VectorSubcoreMesh(
        core_axis_name="c", subcore_axis_name="s",
        num_cores=2, num_subcores=16)

_i32 = jnp.int32
_f32 = jnp.float32


def _iota16():
    return lax.iota(_i32, 16)


def _build_geo_sums(cT, P, sentf, sentc):
    rows_t = NP // 16
    nchunks = rows_t // CHUNK
    gf_t = G3P // 16
    gc_t = GC3P // 16

    @functools.partial(
        pl.kernel,
        out_type=(
            jax.ShapeDtypeStruct((G3P,), _i32),
            jax.ShapeDtypeStruct((GC3P,), _i32),
            jax.ShapeDtypeStruct((NP, 16), _f32),
        ),
        mesh=_sc_mesh(),
        compiler_params=_SC_PARAMS,
        scratch_types=[
            pltpu.VMEM_SHARED((NP, 16), _f32),
            pltpu.VMEM((CHUNK,), _i32),
            pltpu.VMEM((CHUNK,), _i32),
            pltpu.VMEM((CHUNK,), _i32),
            pltpu.VMEM((CHUNK,), _i32),
            pltpu.VMEM((CHUNK,), _i32),
            pltpu.VMEM((CHUNK,), _i32),
            pltpu.VMEM((CHUNK, 16), _f32),
            pltpu.SemaphoreType.DMA,
        ],
    )
    def k(cx_h, cy_h, cz_h, P_h, sentf_h, sentc_h, zacc_h,
          gridf_o, gridc_o, sums_o,
          acc_s, xb, yb, zb, ffb, fcb, valsb, prow, sem):
        cid = lax.axis_index("c")
        sid = lax.axis_index("s")

        @pl.when(cid == 0)
        def _():
            t = sid
            pltpu.sync_copy(sentf_h.at[pl.ds(t * gf_t, gf_t)],
                            gridf_o.at[pl.ds(t * gf_t, gf_t)])
            pltpu.sync_copy(sentc_h.at[pl.ds(t * gc_t, gc_t)],
                            gridc_o.at[pl.ds(t * gc_t, gc_t)])
            pltpu.sync_copy(zacc_h.at[pl.ds(t * rows_t, rows_t)],
                            acc_s.at[pl.ds(t * rows_t, rows_t)])
            plsc.subcore_barrier()

            def load_flat(base):
                pltpu.sync_copy(cx_h.at[pl.ds(base, CHUNK)], xb)
                pltpu.sync_copy(cy_h.at[pl.ds(base, CHUNK)], yb)
                pltpu.sync_copy(cz_h.at[pl.ds(base, CHUNK)], zb)
                for j in range(CHUNK // 16):
                    s = pl.ds(j * 16, 16)
                    xv, yv, zv = xb[s], yb[s], zb[s]
                    ff = (xv * G + yv) * G + zv
                    okf = (ff >= 0) & (ff < G3)
                    ffb[s] = jnp.where(okf, ff, DUMF)
                    xq, yq, zq = xv >> 1, yv >> 1, zv >> 1
                    fc = (xq * GC + yq) * GC + zq
                    okc = (fc >= 0) & (fc < GC3)
                    fcb[s] = jnp.where(okc, fc, DUMC)

            def chunk_b(ci, _):
                base = t * rows_t + ci * CHUNK
                load_flat(base)
                for j in range(CHUNK // 16):
                    s = pl.ds(j * 16, 16)
                    valsb[s] = base + j * 16 + _iota16()
                pltpu.sync_copy(valsb, gridf_o.at[ffb])
                pltpu.sync_copy(valsb, gridc_o.at[fcb])
                return _

            lax.fori_loop(0, nchunks, chunk_b, None)
            plsc.subcore_barrier()

            def chunk_c(ci, _):
                base = t * rows_t + ci * CHUNK
                load_flat(base)
                pltpu.async_copy(gridf_o.at[ffb], valsb, sem).wait()
                pltpu.sync_copy(P_h.at[pl.ds(base, CHUNK)], prow)
                pltpu.sync_copy(prow, acc_s.at[valsb], add=True)
                return _

            lax.fori_loop(0, nchunks, chunk_c, None)
            plsc.subcore_barrier()

            pltpu.sync_copy(acc_s.at[pl.ds(t * rows_t, rows_t)],
                            sums_o.at[pl.ds(t * rows_t, rows_t)])

    zacc = jnp.zeros((NP, 16), _f32)
    return k(cT[0], cT[1], cT[2], P, sentf, sentc, zacc)


def _make_gather(C, K, mode):
    rows_t = NP // 32
    nchunks = rows_t // CHUNK
    gb = 32 if mode == "coarse" else 64
    dum = DUMC if mode == "coarse" else DUMF
    if C <= 96:
        W = 3
    elif C <= 128:
        W = 3
    else:
        W = 2
    W = min(W, K)
    NB = 2 * W

    @functools.partial(
        pl.kernel,
        out_type=jax.ShapeDtypeStruct((K, NP, C), _f32),
        mesh=_sc_mesh(),
        compiler_params=_SC_PARAMS,
        scratch_types=(
            [pltpu.VMEM((CHUNK,), _i32)] * 3
            + [pltpu.VMEM((K, CHUNK), _i32)] * 2
            + [pltpu.VMEM((CHUNK, C), _f32)] * NB
            + [pltpu.SemaphoreType.DMA] * 3
        ),
    )
    def k(cx_h, cy_h, cz_h, grid_h, tbl_h, out_h,
          xb, yb, zb, idx2d, nid2d, *rest):
        bufs = list(rest[:NB])
        semg, semr, semw = rest[NB:]
        cid = lax.axis_index("c")
        sid = lax.axis_index("s")
        wid = sid * 2 + cid
        row0 = wid * rows_t

        def chunk(ci, _):
            base = row0 + ci * CHUNK
            pltpu.sync_copy(cx_h.at[pl.ds(base, CHUNK)], xb)
            pltpu.sync_copy(cy_h.at[pl.ds(base, CHUNK)], yb)
            pltpu.sync_copy(cz_h.at[pl.ds(base, CHUNK)], zb)
            for j in range(CHUNK // 16):
                s = pl.ds(j * 16, 16)
                xv, yv, zv = xb[s], yb[s], zb[s]
                if mode == "fine":
                    bx, by, bz = xv, yv, zv
                elif mode == "coarse":
                    bx, by, bz = xv >> 1, yv >> 1, zv >> 1
                else:
                    bx = (xv >> 1) << 1
                    by = (yv >> 1) << 1
                    bz = (zv >> 1) << 1
                f0 = (bx * gb + by) * gb + bz
                if mode == "child":
                    mok = ((bx >= 0) & (bx < gb) & (by >= 0) & (by < gb)
                           & (bz >= 0) & (bz < gb))
                    for kk in range(K):
                        dx, dy, dz = (kk >> 2) & 1, (kk >> 1) & 1, kk & 1
                        dk = (dx * gb + dy) * gb + dz
                        idx2d[kk, s] = jnp.where(mok, f0 + dk, dum)
                else:
                    mx = {d: (bx + d >= 0) & (bx + d < gb) for d in (-1, 0, 1)}
                    my = {d: (by + d >= 0) & (by + d < gb) for d in (-1, 0, 1)}
                    mz = {d: (bz + d >= 0) & (bz + d < gb) for d in (-1, 0, 1)}
                    for kk in range(K):
                        dx, dy, dz = kk // 9 - 1, (kk // 3) % 3 - 1, kk % 3 - 1
                        dk = (dx * gb + dy) * gb + dz
                        inb = mx[dx] & my[dy] & mz[dz]
                        idx2d[kk, s] = jnp.where(inb, f0 + dk, dum)
            gds = [pltpu.async_copy(grid_h.at[idx2d.at[kk]], nid2d.at[kk],
                                    semg) for kk in range(K)]
            for d in gds:
                d.wait()
            rds, wds = {}, {}
            for kk in range(K):
                if kk >= NB:
                    wds[kk - NB].wait()
                rds[kk] = pltpu.async_copy(tbl_h.at[nid2d.at[kk]],
                                           bufs[kk % NB], semr)
                if kk >= W:
                    kw = kk - W
                    rds[kw].wait()
                    wds[kw] = pltpu.async_copy(
                        bufs[kw % NB], out_h.at[kw, pl.ds(base, CHUNK)], semw)
            for kw in range(max(0, K - W), K):
                rds[kw].wait()
                wds[kw] = pltpu.async_copy(
                    bufs[kw % NB], out_h.at[kw, pl.ds(base, CHUNK)], semw)
            for kw in range(max(0, K - NB), K):
                wds[kw].wait()
            return _

        lax.fori_loop(0, nchunks, chunk, None)

    return k


def _make_up_gather(C):
    rows_t = NP // 32
    nchunks = rows_t // CHUNK

    @functools.partial(
        pl.kernel,
        out_type=jax.ShapeDtypeStruct((NP, C), _f32),
        mesh=_sc_mesh(),
        compiler_params=_SC_PARAMS,
        scratch_types=[
            pltpu.VMEM((CHUNK,), _i32),
            pltpu.VMEM((CHUNK,), _i32),
            pltpu.VMEM((CHUNK,), _i32),
            pltpu.VMEM((CHUNK,), _i32),
            pltpu.VMEM((CHUNK, C), _f32),
            pltpu.SemaphoreType.DMA,
        ],
    )
    def k(cx_h, cy_h, cz_h, tbl_h, out_h, xb, yb, zb, idxb, rows, sem):
        cid = lax.axis_index("c")
        sid = lax.axis_index("s")
        wid = sid * 2 + cid
        row0 = wid * rows_t

        def chunk(ci, _):
            base = row0 + ci * CHUNK
            pltpu.sync_copy(cx_h.at[pl.ds(base, CHUNK)], xb)
            pltpu.sync_copy(cy_h.at[pl.ds(base, CHUNK)], yb)
            pltpu.sync_copy(cz_h.at[pl.ds(base, CHUNK)], zb)
            for j in range(CHUNK // 16):
                s = pl.ds(j * 16, 16)
                oidx = ((xb[s] & 1) * 2 + (yb[s] & 1)) * 2 + (zb[s] & 1)
                idxb[s] = (base + j * 16 + _iota16()) * 8 + oidx
            pltpu.async_copy(tbl_h.at[idxb], rows, sem).wait()
            pltpu.sync_copy(rows, out_h.at[pl.ds(base, CHUNK)])
            return _

        lax.fori_loop(0, nchunks, chunk, None)

    return k


BN = 512
NBLK = NP // BN

_TC_PARAMS = pltpu.CompilerParams(
    dimension_semantics=("parallel", "arbitrary"))
_TC_PARAMS1 = pltpu.CompilerParams(dimension_semantics=("parallel",))


def _bspec_g(BNr, C):
    return pl.BlockSpec((1, BNr, C), lambda i, k: (k, i, 0))


def _bspec_w(C, D):
    return pl.BlockSpec((1, C, D), lambda i, k: (k, 0, 0))


def _bspec_row(BNr, C):
    return pl.BlockSpec((BNr, C), lambda i, k: (i, 0))


def _bspec_vec(D):
    return pl.BlockSpec((1, D), lambda i, k: (0, 0))


def _tc_conv_plain(Gt, W, g, b, relu):
    K, _, C = Gt.shape
    D = W.shape[2]

    def body(G_ref, W_ref, g_ref, b_ref, out_ref):
        k = pl.program_id(1)
        acc = jnp.dot(G_ref[0], W_ref[0], preferred_element_type=_f32)

        @pl.when(k == 0)
        def _():
            out_ref[...] = acc

        @pl.when(k > 0)
        def _():
            out_ref[...] += acc

        @pl.when(k == K - 1)
        def _():
            y = out_ref[...] * g_ref[...] + b_ref[...]
            out_ref[...] = jnp.maximum(y, 0.0) if relu else y

    return pl.pallas_call(
        body,
        grid=(NBLK, K),
        in_specs=[_bspec_g(BN, C), _bspec_w(C, D), _bspec_vec(D), _bspec_vec(D)],
        out_specs=_bspec_row(BN, D),
        out_shape=jax.ShapeDtypeStruct((NP, D), _f32),
        compiler_params=_TC_PARAMS,
    )(Gt, W, g, b)


def _tc_conv_res(Gt, W, g, b, Xd, Wd, gd, bd):
    K, _, C = Gt.shape
    D = W.shape[2]
    Cd = Xd.shape[1]

    def body(G_ref, W_ref, g_ref, b_ref, X_ref, Wd_ref, gd_ref, bd_ref,
             out_ref):
        k = pl.program_id(1)
        acc = jnp.dot(G_ref[0], W_ref[0], preferred_element_type=_f32)

        @pl.when(k == 0)
        def _():
            out_ref[...] = acc

        @pl.when(k > 0)
        def _():
            out_ref[...] += acc

        @pl.when(k == K - 1)
        def _():
            d = jnp.dot(X_ref[...], Wd_ref[...], preferred_element_type=_f32)
            y = (out_ref[...] * g_ref[...] + b_ref[...]
                 + d * gd_ref[...] + bd_ref[...])
            out_ref[...] = jnp.maximum(y, 0.0)

    return pl.pallas_call(
        body,
        grid=(NBLK, K),
        in_specs=[_bspec_g(BN, C), _bspec_w(C, D), _bspec_vec(D),
                  _bspec_vec(D), _bspec_row(BN, Cd),
                  pl.BlockSpec((Cd, D), lambda i, k: (0, 0)),
                  _bspec_vec(D), _bspec_vec(D)],
        out_specs=_bspec_row(BN, D),
        out_shape=jax.ShapeDtypeStruct((NP, D), _f32),
        compiler_params=_TC_PARAMS,
    )(Gt, W, g, b, Xd, Wd, gd, bd)


def _tc_conv_dual(Gt, W1, g1, b1, W2, g2, b2):
    K, _, C = Gt.shape
    D = W1.shape[2]

    def body(G_ref, W1_ref, g1_ref, b1_ref, W2_ref, g2_ref, b2_ref,
             out_ref, acc2):
        k = pl.program_id(1)
        S = G_ref[0]
        V = S / jnp.maximum(S[:, 0:1], 1.0)
        a1 = jnp.dot(V, W1_ref[0], preferred_element_type=_f32)
        a2 = jnp.dot(V, W2_ref[0], preferred_element_type=_f32)

        @pl.when(k == 0)
        def _():
            out_ref[...] = a1
            acc2[...] = a2

        @pl.when(k > 0)
        def _():
            out_ref[...] += a1
            acc2[...] += a2

        @pl.when(k == K - 1)
        def _():
            y1 = jnp.maximum(out_ref[...] * g1_ref[...] + b1_ref[...], 0.0)
            y2 = jnp.maximum(acc2[...] * g2_ref[...] + b2_ref[...], 0.0)
            out_ref[...] = y1 + y2

    return pl.pallas_call(
        body,
        grid=(NBLK, K),
        in_specs=[_bspec_g(BN, C), _bspec_w(C, D), _bspec_vec(D),
                  _bspec_vec(D), _bspec_w(C, D), _bspec_vec(D), _bspec_vec(D)],
        out_specs=_bspec_row(BN, D),
        out_shape=jax.ShapeDtypeStruct((NP, D), _f32),
        scratch_shapes=[pltpu.VMEM((BN, D), _f32)],
        compiler_params=_TC_PARAMS,
    )(Gt, W1, g1, b1, W2, g2, b2)


def _tc_dense_pack(x1, x0, sums, W, bias, g, b):
    C = x0.shape[1]
    D = x1.shape[1]

    def body(x1_ref, x0_ref, s_ref, W_ref, bias_ref, g_ref, b_ref,
             z1_ref, tz_ref):
        f0 = jnp.dot(x0_ref[...], W_ref[...], preferred_element_type=_f32)
        f0 = jnp.maximum((f0 + bias_ref[...]) * g_ref[...] + b_ref[...], 0.0)
        z1 = x1_ref[...] + f0
        z1_ref[...] = z1
        cnt = s_ref[:, 0:1]
        tz_ref[...] = jnp.concatenate(
            [cnt, jnp.zeros((z1.shape[0], 7), _f32), cnt * z1,
             jnp.zeros((z1.shape[0], 8), _f32)], axis=1)

    return pl.pallas_call(
        body,
        grid=(NBLK,),
        in_specs=[pl.BlockSpec((BN, D), lambda i: (i, 0)),
                  pl.BlockSpec((BN, C), lambda i: (i, 0)),
                  pl.BlockSpec((BN, 16), lambda i: (i, 0)),
                  pl.BlockSpec((C, D), lambda i: (0, 0)),
                  pl.BlockSpec((1, D), lambda i: (0, 0)),
                  pl.BlockSpec((1, D), lambda i: (0, 0)),
                  pl.BlockSpec((1, D), lambda i: (0, 0))],
        out_specs=[pl.BlockSpec((BN, D), lambda i: (i, 0)),
                   pl.BlockSpec((BN, D + 16), lambda i: (i, 0))],
        out_shape=[jax.ShapeDtypeStruct((NP, D), _f32),
                   jax.ShapeDtypeStruct((NP, D + 16), _f32)],
        compiler_params=_TC_PARAMS1,
    )(x1, x0, sums, W, bias, g, b)


def _tc_y2c_up(Gz, upW):
    K = Gz.shape[0]
    Ct = Gz.shape[2]
    D = upW.shape[2]

    def body(G_ref, W_ref, out_ref):
        acc = G_ref[0]
        for k in range(1, K):
            acc = acc + G_ref[k]
        y2c = acc[:, 8:136] / jnp.maximum(acc[:, 0:1], 1.0)
        outs = [jnp.dot(y2c, W_ref[k], preferred_element_type=_f32)
                for k in range(8)]
        out_ref[...] = jnp.concatenate(outs, axis=1)

    return pl.pallas_call(
        body,
        grid=(NBLK,),
        in_specs=[pl.BlockSpec((K, BN, Ct), lambda i: (0, i, 0)),
                  pl.BlockSpec((8, 128, D), lambda i: (0, 0, 0))],
        out_specs=pl.BlockSpec((BN, 8 * D), lambda i: (i, 0)),
        out_shape=jax.ShapeDtypeStruct((NP, 8 * D), _f32),
        compiler_params=_TC_PARAMS1,
    )(Gz, upW)


def _tc_upcat(Gu, g, b, x0):
    C = Gu.shape[1]
    C2 = x0.shape[1]

    def body(Gu_ref, g_ref, b_ref, x0_ref, out_ref):
        y = jnp.maximum(Gu_ref[...] * g_ref[...] + b_ref[...], 0.0)
        out_ref[...] = jnp.concatenate([y, x0_ref[...]], axis=1)

    return pl.pallas_call(
        body,
        grid=(NBLK,),
        in_specs=[pl.BlockSpec((BN, C), lambda i: (i, 0)),
                  pl.BlockSpec((1, C), lambda i: (0, 0)),
                  pl.BlockSpec((1, C), lambda i: (0, 0)),
                  pl.BlockSpec((BN, C2), lambda i: (i, 0))],
        out_specs=pl.BlockSpec((BN, C + C2), lambda i: (i, 0)),
        out_shape=jax.ShapeDtypeStruct((NP, C + C2), _f32),
        compiler_params=_TC_PARAMS1,
    )(Gu, g, b, x0)


def _tc_final(G4, W, g, b, y2cat, Wd, gd, bd, z1, Wp, biasp, gp, bp,
              occW, occb):
    K, _, C = G4.shape
    D = W.shape[2]
    Cd = y2cat.shape[1]
    Cp = z1.shape[1]
    Do = occW.shape[1]

    def body(G_ref, W_ref, g_ref, b_ref, yc_ref, Wd_ref, gd_ref, bd_ref,
             z1_ref, Wp_ref, biasp_ref, gp_ref, bp_ref, occW_ref, occb_ref,
             out_ref, acc):
        k = pl.program_id(1)
        a = jnp.dot(G_ref[0], W_ref[0], preferred_element_type=_f32)

        @pl.when(k == 0)
        def _():
            acc[...] = a

        @pl.when(k > 0)
        def _():
            acc[...] += a

        @pl.when(k == K - 1)
        def _():
            d = jnp.dot(yc_ref[...], Wd_ref[...], preferred_element_type=_f32)
            y2f = jnp.maximum(acc[...] * g_ref[...] + b_ref[...]
                              + d * gd_ref[...] + bd_ref[...], 0.0)
            f1 = jnp.dot(z1_ref[...], Wp_ref[...], preferred_element_type=_f32)
            f1 = jnp.maximum((f1 + biasp_ref[...]) * gp_ref[...]
                             + bp_ref[...], 0.0)
            z2 = y2f + f1
            out_ref[...] = jnp.dot(z2, occW_ref[...],
                                   preferred_element_type=_f32) + occb_ref[...]

    return pl.pallas_call(
        body,
        grid=(NBLK, K),
        in_specs=[_bspec_g(BN, C), _bspec_w(C, D), _bspec_vec(D),
                  _bspec_vec(D), _bspec_row(BN, Cd),
                  pl.BlockSpec((Cd, D), lambda i, k: (0, 0)),
                  _bspec_vec(D), _bspec_vec(D), _bspec_row(BN, Cp),
                  pl.BlockSpec((Cp, D), lambda i, k: (0, 0)),
                  _bspec_vec(D), _bspec_vec(D), _bspec_vec(D),
                  pl.BlockSpec((D, Do), lambda i, k: (0, 0)),
                  pl.BlockSpec((1, Do), lambda i, k: (0, 0))],
        out_specs=_bspec_row(BN, Do),
        out_shape=jax.ShapeDtypeStruct((NP, Do), _f32),
        scratch_shapes=[pltpu.VMEM((BN, D), _f32)],
        compiler_params=_TC_PARAMS,
    )(G4, W, g, b, y2cat, Wd, gd, bd, z1, Wp, biasp, gp, bp, occW, occb)


def kernel(z_depth, x_feat_raw, z_feat_raw, coords, pres, vres, params):
    p = params
    n = coords.shape[0]

    cT = jnp.full((3, NP), -64, _i32).at[:, :n].set(coords.T.astype(_i32))
    P = jnp.zeros((NP, 16), _f32)
    P = P.at[:n, 0].set(1.0)
    P = P.at[:n, 1].set(z_depth[:, 0])
    P = P.at[:n, 2:6].set(x_feat_raw)
    sentf = jnp.full((G3P,), SENT, _i32)
    sentc = jnp.full((GC3P,), SENT, _i32)

    def v(x):
        return x.reshape(1, -1)

    c0, c1, c2 = 96, 128, 48
    Wd0 = jnp.zeros((27, 16, c0), _f32).at[:, 1, :].set(p['sd_W'][:, 0, :])
    Wf0 = jnp.zeros((27, 16, c0), _f32).at[:, 2:6, :].set(p['sf_W'])

    gridf, gridc, sums = _build_geo_sums(cT, P, sentf, sentc)

    G0 = _make_gather(16, 27, "fine")(cT[0], cT[1], cT[2], gridf, sums)
    x0 = _tc_conv_dual(G0, Wd0, v(p['sd_g']), v(p['sd_b']),
                       Wf0, v(p['sf_g']), v(p['sf_b']))

    Gd = _make_gather(c0, 8, "child")(cT[0], cT[1], cT[2], gridf, x0)
    x1c = _tc_conv_plain(Gd, p['st_W'], v(p['st_g']), v(p['st_b']), True)

    G1 = _make_gather(c0, 27, "coarse")(cT[0], cT[1], cT[2], gridc, x1c)
    h = _tc_conv_plain(G1, p['r1a_W'], v(p['r1a_g']), v(p['r1a_b']), True)
    G2 = _make_gather(c1, 27, "coarse")(cT[0], cT[1], cT[2], gridc, h)
    x1 = _tc_conv_res(G2, p['r1b_W'], v(p['r1b_g']), v(p['r1b_b']),
                      x1c, p['r1d_W'], v(p['r1d_g']), v(p['r1d_b']))

    z1, Tz = _tc_dense_pack(x1, x0, sums, p['pt0_W'], v(p['pt0_bias']),
                            v(p['pt0_g']), v(p['pt0_b']))
    Gz = _make_gather(c1 + 16, 8, "child")(cT[0], cT[1], cT[2], gridf, Tz)
    Yup = _tc_y2c_up(Gz, p['up_W'])

    Gu = _make_up_gather(c2)(cT[0], cT[1], cT[2], Yup.reshape(8 * NP, c2))
    y2cat = _tc_upcat(Gu, v(p['up_g']), v(p['up_b']), x0)

    G3g = _make_gather(c2 + c0, 27, "fine")(cT[0], cT[1], cT[2], gridf, y2cat)
    h2 = _tc_conv_plain(G3g, p['r2a_W'], v(p['r2a_g']), v(p['r2a_b']), True)
    G4 = _make_gather(c2, 27, "fine")(cT[0], cT[1], cT[2], gridf, h2)
    occW = jnp.zeros((c2, 8), _f32).at[:, 0].set(p['occ_W'][:, 0])
    occb = jnp.zeros((1, 8), _f32).at[0, 0].set(p['occ_b'][0])
    out = _tc_final(G4, p['r2b_W'], v(p['r2b_g']), v(p['r2b_b']),
                    y2cat, p['r2d_W'], v(p['r2d_g']), v(p['r2d_b']),
                    z1, p['pt1_W'], v(p['pt1_bias']), v(p['pt1_g']),
                    v(p['pt1_b']), occW, occb)
    return out[:n, :1]
